# baseline (speedup 1.0000x reference)
import jax
import jax.numpy as jnp
import numpy as np
from jax.experimental import pallas as pl

NODE_DIM = 128
N_GRAPHS = 512
N_NODES = 100000
TILE = 2000
NQ = 32
NR = 16
LOG2 = float(np.log(2.0))


def _fused_body(x_ref, q_ref, r_ref, w1_ref, b1_ref, w2_ref, out_ref):
    i = pl.program_id(0)

    @pl.when(i == 0)
    def _():
        out_ref[...] = jnp.zeros_like(out_ref)

    xb = x_ref[...]
    h = jnp.dot(xb, w1_ref[...], preferred_element_type=jnp.float32)
    h = h + b1_ref[...]
    h = jnp.maximum(h, 0.0) + jnp.log1p(jnp.exp(-jnp.abs(h))) - LOG2
    s = jnp.sum(h * w2_ref[...], axis=1, keepdims=True)

    q = q_ref[0]
    r = r_ref[0]
    a = (jax.lax.broadcasted_iota(jnp.int32, (TILE, NQ), 1) == q)
    b = (jax.lax.broadcasted_iota(jnp.int32, (TILE, NR), 1) == r)
    asb = a.astype(jnp.float32) * s
    contrib = jax.lax.dot_general(
        asb, b.astype(jnp.float32),
        (((0,), (0,)), ((), ())),
        preferred_element_type=jnp.float32)
    out_ref[...] += contrib


@jax.jit
def _run(x, q_r, r_r, W1, b1r, w2r):
    nb = N_NODES // TILE
    out2d = pl.pallas_call(
        _fused_body,
        grid=(nb,),
        in_specs=[
            pl.BlockSpec((TILE, NODE_DIM), lambda i: (i, 0)),
            pl.BlockSpec((1, TILE, 1), lambda i: (i, 0, 0)),
            pl.BlockSpec((1, TILE, 1), lambda i: (i, 0, 0)),
            pl.BlockSpec((NODE_DIM, NODE_DIM), lambda i: (0, 0)),
            pl.BlockSpec((1, NODE_DIM), lambda i: (0, 0)),
            pl.BlockSpec((1, NODE_DIM), lambda i: (0, 0)),
        ],
        out_specs=pl.BlockSpec((NQ, NR), lambda i: (0, 0)),
        out_shape=jax.ShapeDtypeStruct((NQ, NR), jnp.float32),
    )(x, q_r, r_r, W1, b1r, w2r)
    return out2d.reshape(N_GRAPHS, 1)


def kernel(x, batch, W1, b1, W2):
    nb = N_NODES // TILE
    b32 = batch.astype(jnp.int32)
    q_r = (b32 // NR).reshape(nb, TILE, 1)
    r_r = (b32 % NR).reshape(nb, TILE, 1)
    b1r = b1.reshape(1, NODE_DIM)
    w2r = W2.reshape(1, NODE_DIM)
    return _run(x, q_r, r_r, W1, b1r, w2r)


# speedup vs baseline: 3.1298x; 3.1298x over previous
import jax
import jax.numpy as jnp
import numpy as np
from jax.experimental import pallas as pl

NODE_DIM = 128
N_GRAPHS = 512
N_NODES = 100000
TILE = 2000
NQ = 32
NR = 16
LOG2 = float(np.log(2.0))


def _fused_body(x_ref, seg_ref, w1_ref, b1_ref, w2_ref, out_ref):
    i = pl.program_id(0)

    @pl.when(i == 0)
    def _():
        out_ref[...] = jnp.zeros_like(out_ref)

    xb = x_ref[...]
    h = jnp.dot(xb, w1_ref[...], preferred_element_type=jnp.float32)
    h = h + b1_ref[...]
    h = jnp.maximum(h, 0.0) + jnp.log(1.0 + jnp.exp(jnp.minimum(h, -h))) - LOG2
    s = jnp.sum(h * w2_ref[...], axis=1, keepdims=True)

    seg_row = seg_ref[0]
    q_row = seg_row >> 4
    r_row = seg_row & 15
    r_col = r_row.reshape(TILE, 1)
    a = (jax.lax.broadcasted_iota(jnp.int32, (NQ, TILE), 0) == q_row)
    b = (jax.lax.broadcasted_iota(jnp.int32, (TILE, NR), 1) == r_col)
    bs = b.astype(jnp.float32) * s
    contrib = jnp.dot(a.astype(jnp.float32), bs,
                      preferred_element_type=jnp.float32)
    out_ref[...] += contrib


@jax.jit
def _run(x, seg_r, W1, b1r, w2r):
    nb = N_NODES // TILE
    out2d = pl.pallas_call(
        _fused_body,
        grid=(nb,),
        in_specs=[
            pl.BlockSpec((TILE, NODE_DIM), lambda i: (i, 0)),
            pl.BlockSpec((1, 1, TILE), lambda i: (i, 0, 0)),
            pl.BlockSpec((NODE_DIM, NODE_DIM), lambda i: (0, 0)),
            pl.BlockSpec((1, NODE_DIM), lambda i: (0, 0)),
            pl.BlockSpec((1, NODE_DIM), lambda i: (0, 0)),
        ],
        out_specs=pl.BlockSpec((NQ, NR), lambda i: (0, 0)),
        out_shape=jax.ShapeDtypeStruct((NQ, NR), jnp.float32),
    )(x, seg_r, W1, b1r, w2r)
    return out2d.reshape(N_GRAPHS, 1)


def kernel(x, batch, W1, b1, W2):
    nb = N_NODES // TILE
    seg_r = batch.astype(jnp.int32).reshape(nb, 1, TILE)
    b1r = b1.reshape(1, NODE_DIM)
    w2r = W2.reshape(1, NODE_DIM)
    return _run(x, seg_r, W1, b1r, w2r)


# speedup vs baseline: 4.6649x; 1.4905x over previous
import jax
import jax.numpy as jnp
import numpy as np
from jax.experimental import pallas as pl

NODE_DIM = 128
N_GRAPHS = 512
N_NODES = 100000
TILE = 4000
NQ = 32
NR = 16
LOG2 = float(np.log(2.0))


def _fused_body(x_ref, seg_ref, w1_ref, b1_ref, w2_ref, out_ref):
    i = pl.program_id(0)

    @pl.when(i == 0)
    def _():
        out_ref[...] = jnp.zeros_like(out_ref)

    xb = x_ref[...]
    h = jnp.dot(xb, w1_ref[...], preferred_element_type=jnp.float32)
    h = h + b1_ref[...]
    h = jnp.maximum(h, 0.0) + jnp.log(1.0 + jnp.exp(jnp.minimum(h, -h))) - LOG2
    s = jnp.sum(h * w2_ref[...], axis=1, keepdims=True)

    seg_row = seg_ref[0]
    q_row = seg_row >> 4
    r_row = seg_row & 15
    r_col = r_row.reshape(TILE, 1)
    a = (jax.lax.broadcasted_iota(jnp.int32, (NQ, TILE), 0) == q_row)
    b = (jax.lax.broadcasted_iota(jnp.int32, (TILE, NR), 1) == r_col)
    bs = b.astype(jnp.float32) * s
    contrib = jnp.dot(a.astype(jnp.float32), bs,
                      preferred_element_type=jnp.float32)
    out_ref[...] += contrib


@jax.jit
def _run(x, seg_r, W1, b1r, w2r):
    nb = N_NODES // TILE
    out2d = pl.pallas_call(
        _fused_body,
        grid=(nb,),
        in_specs=[
            pl.BlockSpec((TILE, NODE_DIM), lambda i: (i, 0)),
            pl.BlockSpec((1, 1, TILE), lambda i: (i, 0, 0)),
            pl.BlockSpec((NODE_DIM, NODE_DIM), lambda i: (0, 0)),
            pl.BlockSpec((1, NODE_DIM), lambda i: (0, 0)),
            pl.BlockSpec((1, NODE_DIM), lambda i: (0, 0)),
        ],
        out_specs=pl.BlockSpec((NQ, NR), lambda i: (0, 0)),
        out_shape=jax.ShapeDtypeStruct((NQ, NR), jnp.float32),
    )(x, seg_r, W1, b1r, w2r)
    return out2d.reshape(N_GRAPHS, 1)


def kernel(x, batch, W1, b1, W2):
    nb = N_NODES // TILE
    seg_r = batch.astype(jnp.int32).reshape(nb, 1, TILE)
    b1r = b1.reshape(1, NODE_DIM)
    w2r = W2.reshape(1, NODE_DIM)
    return _run(x, seg_r, W1, b1r, w2r)


# speedup vs baseline: 5.1642x; 1.1070x over previous
import jax
import jax.numpy as jnp
import numpy as np
from jax.experimental import pallas as pl

NODE_DIM = 128
N_GRAPHS = 512
N_NODES = 100000
TILE = 10000
NQ = 32
NR = 16
LOG2 = float(np.log(2.0))


def _fused_body(x_ref, seg_ref, w1_ref, b1_ref, w2_ref, out_ref):
    i = pl.program_id(0)

    @pl.when(i == 0)
    def _():
        out_ref[...] = jnp.zeros_like(out_ref)

    xb = x_ref[...]
    h = jnp.dot(xb, w1_ref[...], preferred_element_type=jnp.float32)
    h = h + b1_ref[...]
    h = jnp.maximum(h, 0.0) + jnp.log(1.0 + jnp.exp(jnp.minimum(h, -h))) - LOG2
    s = jnp.sum(h * w2_ref[...], axis=1, keepdims=True)

    seg_row = seg_ref[0]
    q_row = seg_row >> 4
    r_row = seg_row & 15
    r_col = r_row.reshape(TILE, 1)
    a = (jax.lax.broadcasted_iota(jnp.int32, (NQ, TILE), 0) == q_row)
    b = (jax.lax.broadcasted_iota(jnp.int32, (TILE, NR), 1) == r_col)
    bs = b.astype(jnp.float32) * s
    contrib = jnp.dot(a.astype(jnp.float32), bs,
                      preferred_element_type=jnp.float32)
    out_ref[...] += contrib


@jax.jit
def _run(x, seg_r, W1, b1r, w2r):
    nb = N_NODES // TILE
    out2d = pl.pallas_call(
        _fused_body,
        grid=(nb,),
        in_specs=[
            pl.BlockSpec((TILE, NODE_DIM), lambda i: (i, 0)),
            pl.BlockSpec((1, 1, TILE), lambda i: (i, 0, 0)),
            pl.BlockSpec((NODE_DIM, NODE_DIM), lambda i: (0, 0)),
            pl.BlockSpec((1, NODE_DIM), lambda i: (0, 0)),
            pl.BlockSpec((1, NODE_DIM), lambda i: (0, 0)),
        ],
        out_specs=pl.BlockSpec((NQ, NR), lambda i: (0, 0)),
        out_shape=jax.ShapeDtypeStruct((NQ, NR), jnp.float32),
    )(x, seg_r, W1, b1r, w2r)
    return out2d.reshape(N_GRAPHS, 1)


def kernel(x, batch, W1, b1, W2):
    nb = N_NODES // TILE
    seg_r = batch.astype(jnp.int32).reshape(nb, 1, TILE)
    b1r = b1.reshape(1, NODE_DIM)
    w2r = W2.reshape(1, NODE_DIM)
    return _run(x, seg_r, W1, b1r, w2r)


# speedup vs baseline: 5.1984x; 1.0066x over previous
import jax
import jax.numpy as jnp
import numpy as np
from jax.experimental import pallas as pl

NODE_DIM = 128
N_GRAPHS = 512
N_NODES = 100000
TILE = 20000
NQ = 32
NR = 16
LOG2 = float(np.log(2.0))


def _fused_body(x_ref, seg_ref, w1_ref, b1_ref, w2_ref, out_ref):
    i = pl.program_id(0)

    @pl.when(i == 0)
    def _():
        out_ref[...] = jnp.zeros_like(out_ref)

    xb = x_ref[...]
    h = jnp.dot(xb, w1_ref[...], preferred_element_type=jnp.float32)
    h = h + b1_ref[...]
    h = jnp.maximum(h, 0.0) + jnp.log(1.0 + jnp.exp(jnp.minimum(h, -h))) - LOG2
    s = jnp.sum(h * w2_ref[...], axis=1, keepdims=True)

    seg_row = seg_ref[0]
    q_row = seg_row >> 4
    r_row = seg_row & 15
    r_col = r_row.reshape(TILE, 1)
    a = (jax.lax.broadcasted_iota(jnp.int32, (NQ, TILE), 0) == q_row)
    b = (jax.lax.broadcasted_iota(jnp.int32, (TILE, NR), 1) == r_col)
    bs = b.astype(jnp.float32) * s
    contrib = jnp.dot(a.astype(jnp.float32), bs,
                      preferred_element_type=jnp.float32)
    out_ref[...] += contrib


@jax.jit
def _run(x, seg_r, W1, b1r, w2r):
    nb = N_NODES // TILE
    out2d = pl.pallas_call(
        _fused_body,
        grid=(nb,),
        in_specs=[
            pl.BlockSpec((TILE, NODE_DIM), lambda i: (i, 0)),
            pl.BlockSpec((1, 1, TILE), lambda i: (i, 0, 0)),
            pl.BlockSpec((NODE_DIM, NODE_DIM), lambda i: (0, 0)),
            pl.BlockSpec((1, NODE_DIM), lambda i: (0, 0)),
            pl.BlockSpec((1, NODE_DIM), lambda i: (0, 0)),
        ],
        out_specs=pl.BlockSpec((NQ, NR), lambda i: (0, 0)),
        out_shape=jax.ShapeDtypeStruct((NQ, NR), jnp.float32),
    )(x, seg_r, W1, b1r, w2r)
    return out2d.reshape(N_GRAPHS, 1)


def kernel(x, batch, W1, b1, W2):
    nb = N_NODES // TILE
    seg_r = batch.astype(jnp.int32).reshape(nb, 1, TILE)
    b1r = b1.reshape(1, NODE_DIM)
    w2r = W2.reshape(1, NODE_DIM)
    return _run(x, seg_r, W1, b1r, w2r)
